# baseline (speedup 1.0000x reference)
import functools

import jax
import jax.numpy as jnp
from jax import lax
from jax.experimental import pallas as pl
from jax.experimental.pallas import tpu as pltpu
from jax.experimental.pallas import tpu_sc as plsc

NUM_CLASSES = 1_000_000
HIDDEN = 64
BATCH = 16384

_NC = 2
_NS = 16
_NW = _NC * _NS

_B_PER_W = BATCH // _NW
_CHUNK = 128
_NCHUNK = _B_PER_W // _CHUNK


@functools.partial(
    pl.kernel,
    out_type=jax.ShapeDtypeStruct((_NW, _NCHUNK, _CHUNK, HIDDEN), jnp.float32),
    mesh=plsc.VectorSubcoreMesh(core_axis_name="c", subcore_axis_name="s"),
    scratch_types=[
        pltpu.VMEM((_NCHUNK, _CHUNK), jnp.int32),
        pltpu.VMEM((_NCHUNK, _CHUNK, HIDDEN), jnp.float32),
        pltpu.SemaphoreType.DMA,
    ],
    compiler_params=pltpu.CompilerParams(use_tc_tiling_on_sc=False),
)
def _gather_kernel(table_hbm, idx_hbm, out_hbm, idx_v, rows_v, sem):
    wid = lax.axis_index("s") * _NC + lax.axis_index("c")
    pltpu.sync_copy(idx_hbm.at[wid], idx_v)
    copies = []
    for j in range(_NCHUNK):
        copies.append(
            pltpu.async_copy(table_hbm.at[idx_v.at[j]], rows_v.at[j], sem)
        )
    for c in copies:
        c.wait()
    pltpu.sync_copy(rows_v, out_hbm.at[wid])


def kernel(labels, embedding_table):
    idx = labels.astype(jnp.int32).reshape(_NW, _NCHUNK, _CHUNK)
    out = _gather_kernel(embedding_table, idx)
    return out.reshape(BATCH, HIDDEN)


# speedup vs baseline: 1.6781x; 1.6781x over previous
import functools

import jax
import jax.numpy as jnp
from jax import lax
from jax.experimental import pallas as pl
from jax.experimental.pallas import tpu as pltpu
from jax.experimental.pallas import tpu_sc as plsc

NUM_CLASSES = 1_000_000
HIDDEN = 64
BATCH = 16384

_NC = 2
_NS = 16
_NW = _NC * _NS

_B_PER_W = BATCH // _NW
_CHUNK = 32
_NCHUNK = _B_PER_W // _CHUNK


@functools.partial(
    pl.kernel,
    out_type=jax.ShapeDtypeStruct((BATCH, HIDDEN), jnp.float32),
    mesh=plsc.VectorSubcoreMesh(core_axis_name="c", subcore_axis_name="s"),
    scratch_types=[
        pltpu.VMEM((_B_PER_W,), jnp.int32),
        pltpu.VMEM((_CHUNK, HIDDEN), jnp.float32),
        pltpu.SemaphoreType.DMA,
    ],
)
def _gather_kernel(table_hbm, idx_hbm, out_hbm, labels_v, rowbuf, sem):
    wid = lax.axis_index("s") * _NC + lax.axis_index("c")
    base = wid * _B_PER_W
    pltpu.sync_copy(idx_hbm.at[pl.ds(base, _B_PER_W)], labels_v)

    def do_chunk(j, _):
        copies = []
        for h in range(_CHUNK // 16):
            lvec = labels_v[pl.ds(j * _CHUNK + h * 16, 16)]
            for g in range(16):
                i = h * 16 + g
                lab = lvec[g]
                copies.append(
                    pltpu.async_copy(table_hbm.at[lab], rowbuf.at[i], sem))
        for c in copies:
            c.wait()
        pltpu.sync_copy(rowbuf, out_hbm.at[pl.ds(base + j * _CHUNK, _CHUNK)])
        return 0

    lax.fori_loop(0, _NCHUNK, do_chunk, 0)


def kernel(labels, embedding_table):
    return _gather_kernel(embedding_table, labels.astype(jnp.int32))
